# Initial kernel scaffold; baseline (speedup 1.0000x reference)
import jax
import jax.numpy as jnp
from jax.experimental import pallas as pl


def kernel(input, batch):
    raise NotImplementedError("write your pallas kernel here")


# speedup vs baseline: 2.1046x; 2.1046x over previous
import jax
import jax.numpy as jnp
from jax.experimental import pallas as pl
from jax.experimental.pallas import tpu as pltpu

N_ROWS = 100000
N_SEG = 64
D = 128
BLK = 1000
GRID = N_ROWS // BLK


def _body(x_ref, b_ref, y_ref, out_ref, sums_ref, cnts_ref):
    i = pl.program_id(0)

    @pl.when(i == 0)
    def _init():
        sums_ref[...] = jnp.zeros_like(sums_ref)
        cnts_ref[...] = jnp.zeros_like(cnts_ref)

    x = x_ref[...]
    y_ref[...] = x
    norms = jnp.sqrt(jnp.sum(x * x, axis=1, keepdims=True))
    b = b_ref[0]
    bt = b.reshape(BLK, 1)
    seg = jax.lax.broadcasted_iota(jnp.int32, (BLK, N_SEG), 1)
    onehot = bt == seg
    sums_ref[...] += jnp.sum(jnp.where(onehot, norms, 0.0), axis=0, keepdims=True)
    cnts_ref[...] += jnp.sum(onehot.astype(jnp.float32), axis=0, keepdims=True)

    @pl.when(i == GRID - 1)
    def _fin():
        bs = b[0, BLK - 1]
        w = sums_ref[...] / cnts_ref[...]
        mask = jax.lax.broadcasted_iota(jnp.int32, (1, N_SEG), 1) < bs
        nm = jnp.sum(jnp.where(mask, w, 0.0)) / (bs + 1).astype(jnp.float32)
        out_ref[...] = nm.reshape(1, 1)


def kernel(input, batch):
    batch3 = batch.reshape(GRID, 1, BLK)
    y, nm = pl.pallas_call(
        _body,
        grid=(GRID,),
        in_specs=[
            pl.BlockSpec((BLK, D), lambda i: (i, 0)),
            pl.BlockSpec((1, 1, BLK), lambda i: (i, 0, 0)),
        ],
        out_specs=[
            pl.BlockSpec((BLK, D), lambda i: (i, 0)),
            pl.BlockSpec((1, 1), lambda i: (0, 0)),
        ],
        out_shape=[
            jax.ShapeDtypeStruct((N_ROWS, D), jnp.float32),
            jax.ShapeDtypeStruct((1, 1), jnp.float32),
        ],
        scratch_shapes=[
            pltpu.VMEM((1, N_SEG), jnp.float32),
            pltpu.VMEM((1, N_SEG), jnp.float32),
        ],
    )(input, batch3)
    return y, nm.reshape(())
